# baseline (speedup 1.0000x reference)
import functools

import jax
import jax.numpy as jnp
from jax import lax
from jax.experimental import pallas as pl
from jax.experimental.pallas import tpu as pltpu
from jax.experimental.pallas import tpu_sc as plsc

_BIGF = float(2**28)
_QB = 16
_CHUNK = 1024
_UNROLL = 4


def _search_body(n, pos_ref, tx_ref, ty_ref, out_ref):
    nch = tx_ref.shape[0]
    qx = pos_ref[:, 0].reshape(_QB, 1, 1)
    qy = pos_ref[:, 1].reshape(_QB, 1, 1)
    lane = (
        lax.broadcasted_iota(jnp.int32, (1, 8, 128), 1) * 128
        + lax.broadcasted_iota(jnp.int32, (1, 8, 128), 2)
    ).astype(jnp.float32)

    def step(c, acc):
        for u in range(_UNROLL):
            cc = c * _UNROLL + u
            hit = (tx_ref[cc][None] == qx) | (ty_ref[cc][None] == qy)
            idxf = lane + (cc * _CHUNK).astype(jnp.float32)
            acc = jnp.minimum(acc, jnp.where(hit, idxf, _BIGF))
        return acc

    acc0 = jnp.full((_QB, 8, 128), _BIGF, jnp.float32)
    acc = lax.fori_loop(0, nch // _UNROLL, step, acc0)
    mi = jnp.min(acc, axis=(1, 2)).astype(jnp.int32)
    out_ref[:, 0] = jnp.where(mi >= n, 0, mi)


def _tc_search(positions, obs_pos):
    n = obs_pos.shape[0]
    b = positions.shape[0]
    step_rows = _CHUNK * _UNROLL
    nch = _UNROLL * ((n + step_rows - 1) // step_rows)
    npad = nch * _CHUNK
    tx = jnp.pad(obs_pos[:, 0], (0, npad - n)).reshape(nch, 8, 128)
    ty = jnp.pad(obs_pos[:, 1], (0, npad - n)).reshape(nch, 8, 128)
    minidx = pl.pallas_call(
        functools.partial(_search_body, n),
        grid=(b // _QB,),
        in_specs=[
            pl.BlockSpec((_QB, 2), lambda q: (q, 0)),
            pl.BlockSpec((nch, 8, 128), lambda q: (0, 0, 0)),
            pl.BlockSpec((nch, 8, 128), lambda q: (0, 0, 0)),
        ],
        out_specs=pl.BlockSpec((_QB, 1), lambda q: (q, 0)),
        out_shape=jax.ShapeDtypeStruct((b, 1), jnp.int32),
    )(positions, tx, ty)
    return minidx.reshape(b)


def _sc_gather(table, idx):
    b = idx.shape[0]
    info = plsc.get_sparse_core_info()
    nw = info.num_cores * info.num_subcores
    bpw = b // nw
    mesh = plsc.VectorSubcoreMesh(core_axis_name="c", subcore_axis_name="s")

    @functools.partial(
        pl.kernel,
        mesh=mesh,
        out_type=jax.ShapeDtypeStruct((b, 128), jnp.float32),
        scratch_types=[
            pltpu.VMEM((bpw,), jnp.int32),
            pltpu.VMEM((bpw, 128), jnp.float32),
            pltpu.SemaphoreType.DMA,
        ],
    )
    def gk(table_hbm, idx_hbm, out_hbm, idx_v, rows_v, sem):
        wid = lax.axis_index("s") * info.num_cores + lax.axis_index("c")
        base = wid * bpw
        pltpu.sync_copy(idx_hbm.at[pl.ds(base, bpw)], idx_v)
        pltpu.async_copy(table_hbm.at[idx_v], rows_v, sem).wait()
        pltpu.sync_copy(rows_v, out_hbm.at[pl.ds(base, bpw)])

    return gk(table, idx)


def kernel(positions, obs_pos, zks_prior):
    idx = _tc_search(positions, obs_pos)
    d = zks_prior.shape[1]
    zpad = jnp.pad(zks_prior, ((0, 0), (0, 128 - d)))
    rows = _sc_gather(zpad, idx)
    return rows[:, :d, None, None]


# speedup vs baseline: 2.1647x; 2.1647x over previous
import functools

import jax
import jax.numpy as jnp
from jax import lax
from jax.experimental import pallas as pl
from jax.experimental.pallas import tpu as pltpu
from jax.experimental.pallas import tpu_sc as plsc

_BIGF = float(2**28)
_QB = 512
_CHUNK = 1024
_UNROLL = 4


def _search_body(n, pos_ref, tx_ref, ty_ref, out_ref):
    nch = tx_ref.shape[0]
    qx = pos_ref[:, 0].reshape(_QB, 1, 1)
    qy = pos_ref[:, 1].reshape(_QB, 1, 1)
    lane = (
        lax.broadcasted_iota(jnp.int32, (1, 8, 128), 1) * 128
        + lax.broadcasted_iota(jnp.int32, (1, 8, 128), 2)
    ).astype(jnp.float32)

    def step(c, acc):
        for u in range(_UNROLL):
            cc = c * _UNROLL + u
            hit = (tx_ref[cc][None] == qx) | (ty_ref[cc][None] == qy)
            idxf = lane + (cc * _CHUNK).astype(jnp.float32)
            acc = jnp.minimum(acc, jnp.where(hit, idxf, _BIGF))
        return acc

    acc0 = jnp.full((_QB, 8, 128), _BIGF, jnp.float32)
    acc = lax.fori_loop(0, nch // _UNROLL, step, acc0)
    mi = jnp.min(acc, axis=(1, 2)).astype(jnp.int32)
    out_ref[:, 0] = jnp.where(mi >= n, 0, mi)


def _tc_search(positions, obs_pos):
    n = obs_pos.shape[0]
    b = positions.shape[0]
    step_rows = _CHUNK * _UNROLL
    nch = _UNROLL * ((n + step_rows - 1) // step_rows)
    npad = nch * _CHUNK
    tx = jnp.pad(obs_pos[:, 0], (0, npad - n)).reshape(nch, 8, 128)
    ty = jnp.pad(obs_pos[:, 1], (0, npad - n)).reshape(nch, 8, 128)
    minidx = pl.pallas_call(
        functools.partial(_search_body, n),
        grid=(b // _QB,),
        in_specs=[
            pl.BlockSpec((_QB, 2), lambda q: (q, 0)),
            pl.BlockSpec((nch, 8, 128), lambda q: (0, 0, 0)),
            pl.BlockSpec((nch, 8, 128), lambda q: (0, 0, 0)),
        ],
        out_specs=pl.BlockSpec((_QB, 1), lambda q: (q, 0)),
        out_shape=jax.ShapeDtypeStruct((b, 1), jnp.int32),
    )(positions, tx, ty)
    return minidx.reshape(b)


def _sc_gather(table, idx):
    b = idx.shape[0]
    info = plsc.get_sparse_core_info()
    nw = info.num_cores * info.num_subcores
    bpw = b // nw
    mesh = plsc.VectorSubcoreMesh(core_axis_name="c", subcore_axis_name="s")

    @functools.partial(
        pl.kernel,
        mesh=mesh,
        out_type=jax.ShapeDtypeStruct((b, 128), jnp.float32),
        scratch_types=[
            pltpu.VMEM((bpw,), jnp.int32),
            pltpu.VMEM((bpw, 128), jnp.float32),
            pltpu.SemaphoreType.DMA,
        ],
    )
    def gk(table_hbm, idx_hbm, out_hbm, idx_v, rows_v, sem):
        wid = lax.axis_index("s") * info.num_cores + lax.axis_index("c")
        base = wid * bpw
        pltpu.sync_copy(idx_hbm.at[pl.ds(base, bpw)], idx_v)
        pltpu.async_copy(table_hbm.at[idx_v], rows_v, sem).wait()
        pltpu.sync_copy(rows_v, out_hbm.at[pl.ds(base, bpw)])

    return gk(table, idx)


def kernel(positions, obs_pos, zks_prior):
    d = zks_prior.shape[1]
    zpad = jnp.pad(zks_prior, ((0, 0), (0, 128 - d)))
    idx = _tc_search(positions, obs_pos)
    rows = _sc_gather(zpad, idx)
    return rows[:, :d, None, None]


# speedup vs baseline: 2.4009x; 1.1091x over previous
import functools

import jax
import jax.numpy as jnp
from jax import lax
from jax.experimental import pallas as pl
from jax.experimental.pallas import tpu as pltpu
from jax.experimental.pallas import tpu_sc as plsc

_BIGF = float(2**28)
_QB = 512
_CHUNK = 1024
_UNROLL = 4


def _search_body(n, pos_ref, tx_ref, ty_ref, out_ref):
    nch = tx_ref.shape[0]
    qx = pos_ref[:, 0].reshape(_QB, 1, 1)
    qy = pos_ref[:, 1].reshape(_QB, 1, 1)
    lane = (
        lax.broadcasted_iota(jnp.int32, (1, 8, 128), 1) * 128
        + lax.broadcasted_iota(jnp.int32, (1, 8, 128), 2)
    ).astype(jnp.float32)

    def step(c, acc):
        for u in range(_UNROLL):
            cc = nch - 1 - (c * _UNROLL + u)
            hit = (tx_ref[cc][None] == qx) | (ty_ref[cc][None] == qy)
            idxf = lane + (cc * _CHUNK).astype(jnp.float32)
            acc = jnp.where(hit, idxf, acc)
        return acc

    acc0 = jnp.full((_QB, 8, 128), _BIGF, jnp.float32)
    acc = lax.fori_loop(0, nch // _UNROLL, step, acc0)
    mi = jnp.min(acc, axis=(1, 2)).astype(jnp.int32)
    out_ref[:, 0] = jnp.where(mi >= n, 0, mi)


def _tc_search(positions, obs_pos):
    n = obs_pos.shape[0]
    b = positions.shape[0]
    step_rows = _CHUNK * _UNROLL
    nch = _UNROLL * ((n + step_rows - 1) // step_rows)
    npad = nch * _CHUNK
    tx = jnp.pad(obs_pos[:, 0], (0, npad - n)).reshape(nch, 8, 128)
    ty = jnp.pad(obs_pos[:, 1], (0, npad - n)).reshape(nch, 8, 128)
    minidx = pl.pallas_call(
        functools.partial(_search_body, n),
        grid=(b // _QB,),
        in_specs=[
            pl.BlockSpec((_QB, 2), lambda q: (q, 0)),
            pl.BlockSpec((nch, 8, 128), lambda q: (0, 0, 0)),
            pl.BlockSpec((nch, 8, 128), lambda q: (0, 0, 0)),
        ],
        out_specs=pl.BlockSpec((_QB, 1), lambda q: (q, 0)),
        out_shape=jax.ShapeDtypeStruct((b, 1), jnp.int32),
    )(positions, tx, ty)
    return minidx.reshape(b)


def _sc_gather(table, idx):
    b = idx.shape[0]
    info = plsc.get_sparse_core_info()
    nw = info.num_cores * info.num_subcores
    bpw = b // nw
    mesh = plsc.VectorSubcoreMesh(core_axis_name="c", subcore_axis_name="s")

    @functools.partial(
        pl.kernel,
        mesh=mesh,
        out_type=jax.ShapeDtypeStruct((b, 128), jnp.float32),
        scratch_types=[
            pltpu.VMEM((bpw,), jnp.int32),
            pltpu.VMEM((bpw, 128), jnp.float32),
            pltpu.SemaphoreType.DMA,
        ],
    )
    def gk(table_hbm, idx_hbm, out_hbm, idx_v, rows_v, sem):
        wid = lax.axis_index("s") * info.num_cores + lax.axis_index("c")
        base = wid * bpw
        pltpu.sync_copy(idx_hbm.at[pl.ds(base, bpw)], idx_v)
        pltpu.async_copy(table_hbm.at[idx_v], rows_v, sem).wait()
        pltpu.sync_copy(rows_v, out_hbm.at[pl.ds(base, bpw)])

    return gk(table, idx)


def kernel(positions, obs_pos, zks_prior):
    d = zks_prior.shape[1]
    zpad = jnp.pad(zks_prior, ((0, 0), (0, 128 - d)))
    idx = _tc_search(positions, obs_pos)
    rows = _sc_gather(zpad, idx)
    return rows[:, :d, None, None]
